# baseline (speedup 1.0000x reference)
import dataclasses
import functools

import jax
import jax.numpy as jnp
from jax import lax
from jax.experimental import pallas as pl
from jax.experimental.pallas import tpu as pltpu
from jax.experimental.pallas import tpu_sc as plsc

N = 10000
E = 320000
C = 128
H = 64
NCORE = 2
NSUB = 16
LANES = 16

EPS = E // NSUB
B = 32
NCHUNK = EPS // B
ROWCH = 16
NROWCH = N // ROWCH

MBLK = 400


def _ycat_body(xr_ref, xi_ref, w_ref, o_ref):
    w = w_ref[0]
    a = jnp.dot(xr_ref[...], w, preferred_element_type=jnp.float32)
    b = jnp.dot(xi_ref[...], w, preferred_element_type=jnp.float32)
    o_ref[0, :, 0:192] = a
    o_ref[0, :, 192:384] = b


def _project(x_real, x_imag, wsel):
    return pl.pallas_call(
        _ycat_body,
        grid=(NCORE, N // MBLK),
        in_specs=[
            pl.BlockSpec((MBLK, C), lambda c, n: (n, 0)),
            pl.BlockSpec((MBLK, C), lambda c, n: (n, 0)),
            pl.BlockSpec((1, C, 3 * H), lambda c, n: (c, 0, 0)),
        ],
        out_specs=pl.BlockSpec((1, MBLK, 6 * H), lambda c, n: (c, n, 0)),
        out_shape=jax.ShapeDtypeStruct((NCORE, N, 6 * H), jnp.float32),
    )(x_real, x_imag, wsel)


_sc_mesh = plsc.VectorSubcoreMesh(core_axis_name="c", subcore_axis_name="s")

_sc_params = pltpu.CompilerParams()
if "needs_layout_passes" in pltpu.CompilerParams.__dataclass_fields__:
    _sc_params = dataclasses.replace(_sc_params, needs_layout_passes=False)


@functools.partial(
    pl.kernel,
    out_type=jax.ShapeDtypeStruct((NCORE * N, C), jnp.float32),
    mesh=_sc_mesh,
    scratch_types=[
        pltpu.VMEM((B, 8), jnp.float32),
        pltpu.VMEM((B, 8), jnp.float32),
        pltpu.VMEM((B,), jnp.int32),
        pltpu.VMEM((B,), jnp.int32),
        pltpu.VMEM((B,), jnp.int32),
        pltpu.VMEM((B,), jnp.int32),
        pltpu.VMEM((B,), jnp.int32),
        pltpu.VMEM((B,), jnp.int32),
        pltpu.VMEM((B, 6 * H), jnp.float32),
        pltpu.VMEM((B, 6 * H), jnp.float32),
        pltpu.VMEM((B, C), jnp.float32),
        pltpu.VMEM((B, C), jnp.float32),
        pltpu.VMEM((ROWCH, C), jnp.float32),
        pltpu.VMEM_SHARED((N, C), jnp.float32),
        pltpu.SemaphoreType.DMA,
        pltpu.SemaphoreType.DMA,
        pltpu.SemaphoreType.DMA,
        pltpu.SemaphoreType.DMA,
    ],
    compiler_params=_sc_params,
)
def _sc_spmm(ycat_hbm, rec_hbm, out_hbm,
             recv0, recv1, srcv0, srcv1, dstv0, dstv1, dstv2, dstv3,
             rows0, rows1, outv0, outv1, zbuf, acc,
             gsem0, gsem1, ssem0, ssem1):
    c = lax.axis_index("c")
    s = lax.axis_index("s")

    recvs = [recv0, recv1]
    srcvs = [srcv0, srcv1]
    dstvs = [dstv0, dstv1, dstv2, dstv3]
    rowss = [rows0, rows1]
    outvs = [outv0, outv1]
    gsems = [gsem0, gsem1]
    ssems = [ssem0, ssem1]

    zero16 = jnp.zeros((LANES,), jnp.float32)

    @pl.loop(0, ROWCH)
    def _(r):
        for kk in range(C // LANES):
            zbuf[r, pl.ds(kk * LANES, LANES)] = zero16

    @pl.loop(s, NROWCH, step=NSUB)
    def _(ch):
        pltpu.sync_copy(zbuf, acc.at[pl.ds(ch * ROWCH, ROWCH)])

    plsc.subcore_barrier()

    iota16 = jax.lax.iota(jnp.int32, LANES)
    rowh = [iota16 + (h * LANES) for h in range(B // LANES)]
    col6 = jnp.full((LANES,), 6, dtype=jnp.int32)
    col7 = jnp.full((LANES,), 7, dtype=jnp.int32)
    coffv = jnp.full((LANES,), c * N, dtype=jnp.int32)
    icol = [jnp.full((LANES,), i, dtype=jnp.int32) for i in range(6)]
    base0 = s * EPS

    def load_chunk(t, recv_b):
        pltpu.sync_copy(rec_hbm.at[pl.ds(base0 + t * B, B)], recv_b)

    def extract(recv_b, srcv_b, dstv_b):
        for h in range(B // LANES):
            sv = plsc.load_gather(recv_b, [rowh[h], col6]).astype(jnp.int32)
            dv = plsc.load_gather(recv_b, [rowh[h], col7]).astype(jnp.int32)
            srcv_b[pl.ds(h * LANES, LANES)] = sv + coffv
            dstv_b[pl.ds(h * LANES, LANES)] = dv

    def gather_issue(srcv_b, rows_b, sem):
        pltpu.async_copy(ycat_hbm.at[srcv_b], rows_b, sem)

    def gather_wait(srcv_b, rows_b, sem):
        pltpu.make_async_copy(ycat_hbm.at[srcv_b], rows_b, sem).wait()

    def scatter_issue(outv_b, dstv_b, sem):
        pltpu.async_copy(outv_b, acc.at[dstv_b], sem, add=True)

    def scatter_wait(outv_b, dstv_b, sem):
        pltpu.make_async_copy(outv_b, acc.at[dstv_b], sem).wait()

    def compute(recv_b, rows_b, outv_b):
        @plsc.parallel_loop(0, B, 1, unroll=3)
        def _(j):
            jvec = jnp.full((LANES,), j, dtype=jnp.int32)
            w0 = plsc.load_gather(recv_b, [jvec, icol[0]])
            w1 = plsc.load_gather(recv_b, [jvec, icol[1]])
            w2 = plsc.load_gather(recv_b, [jvec, icol[2]])
            w3 = plsc.load_gather(recv_b, [jvec, icol[3]])
            w4 = plsc.load_gather(recv_b, [jvec, icol[4]])
            w5 = plsc.load_gather(recv_b, [jvec, icol[5]])
            for k in range(H // LANES):
                o = k * LANES
                yr0 = rows_b[j, pl.ds(o, LANES)]
                yr1 = rows_b[j, pl.ds(H + o, LANES)]
                yr2 = rows_b[j, pl.ds(2 * H + o, LANES)]
                yi0 = rows_b[j, pl.ds(3 * H + o, LANES)]
                yi1 = rows_b[j, pl.ds(4 * H + o, LANES)]
                yi2 = rows_b[j, pl.ds(5 * H + o, LANES)]
                re = (w0 * yr0 + w1 * yr1 + w2 * yr2
                      - w3 * yi0 - w4 * yi1 - w5 * yi2)
                im = (w3 * yr0 + w4 * yr1 + w5 * yr2
                      + w0 * yi0 + w1 * yi1 + w2 * yi2)
                outv_b[j, pl.ds(o, LANES)] = re
                outv_b[j, pl.ds(H + o, LANES)] = im

    for t in range(2):
        load_chunk(t, recvs[t])
        extract(recvs[t], srcvs[t], dstvs[t])
        gather_issue(srcvs[t], rowss[t], gsems[t])

    @pl.loop(0, NCHUNK - 1, step=4)
    def _(t0):
        for b4 in range(4):
            t = t0 + b4
            b2 = b4 % 2
            gather_wait(srcvs[b2], rowss[b2], gsems[b2])

            @pl.when(t >= 2)
            def _():
                scatter_wait(outvs[b2], dstvs[(b4 + 2) % 4], ssems[b2])

            compute(recvs[b2], rowss[b2], outvs[b2])
            scatter_issue(outvs[b2], dstvs[b4], ssems[b2])
            load_chunk(t + 2, recvs[b2])
            extract(recvs[b2], srcvs[b2], dstvs[(b4 + 2) % 4])
            gather_issue(srcvs[b2], rowss[b2], gsems[b2])

    gather_wait(srcvs[0], rowss[0], gsems[0])
    scatter_wait(outvs[0], dstvs[2], ssems[0])
    compute(recvs[0], rowss[0], outvs[0])
    scatter_issue(outvs[0], dstvs[0], ssems[0])
    gather_wait(srcvs[1], rowss[1], gsems[1])
    scatter_wait(outvs[1], dstvs[3], ssems[1])
    scatter_wait(outvs[0], dstvs[0], ssems[0])

    plsc.subcore_barrier()

    @pl.loop(s, NROWCH, step=NSUB)
    def _(ch):
        r0 = ch * ROWCH
        pltpu.sync_copy(acc.at[pl.ds(r0, ROWCH)],
                        out_hbm.at[pl.ds(c * N + r0, ROWCH)])


def kernel(x_real, x_imag, edge_index, l_real_w, l_imag_w, weight, bias):
    dst = edge_index[0]
    src = edge_index[1]

    wsel = jnp.stack([
        jnp.concatenate([weight[0][:, :H], weight[1][:, :H], weight[2][:, :H]], axis=1),
        jnp.concatenate([weight[0][:, H:], weight[1][:, H:], weight[2][:, H:]], axis=1),
    ])

    ycat = _project(x_real, x_imag, wsel).reshape(NCORE * N, 6 * H)

    rec = jnp.concatenate(
        [l_real_w.T, l_imag_w.T,
         src.astype(jnp.float32)[:, None],
         dst.astype(jnp.float32)[:, None]], axis=1)
    rec = jnp.concatenate([rec, jnp.zeros((B, 8), jnp.float32)], axis=0)

    out = _sc_spmm(ycat, rec).reshape(NCORE, N, C)

    real = jnp.concatenate([out[0, :, :H], out[1, :, :H]], axis=1) + bias
    imag = jnp.concatenate([out[0, :, H:], out[1, :, H:]], axis=1) + bias
    return (real, imag)


# speedup vs baseline: 1.4538x; 1.4538x over previous
import dataclasses
import functools

import jax
import jax.numpy as jnp
from jax import lax
from jax.experimental import pallas as pl
from jax.experimental.pallas import tpu as pltpu
from jax.experimental.pallas import tpu_sc as plsc

N = 10000
E = 320000
C = 128
H = 64
NCORE = 2
NSUB = 16
LANES = 16

EPS = E // NSUB
B = 32
NCHUNK = EPS // B
ZROW = 8
NZCH = N // ZROW
DROW = 40
NDCH = N // DROW

MBLK = 400


def _ycat_body(xr_ref, xi_ref, w_ref, o_ref):
    w = w_ref[0]
    a = jnp.dot(xr_ref[...], w, preferred_element_type=jnp.float32)
    b = jnp.dot(xi_ref[...], w, preferred_element_type=jnp.float32)
    o_ref[0, :, 0:192] = a
    o_ref[0, :, 192:384] = b


def _project(x_real, x_imag, wsel):
    return pl.pallas_call(
        _ycat_body,
        grid=(NCORE, N // MBLK),
        in_specs=[
            pl.BlockSpec((MBLK, C), lambda c, n: (n, 0)),
            pl.BlockSpec((MBLK, C), lambda c, n: (n, 0)),
            pl.BlockSpec((1, C, 3 * H), lambda c, n: (c, 0, 0)),
        ],
        out_specs=pl.BlockSpec((1, MBLK, 6 * H), lambda c, n: (c, n, 0)),
        out_shape=jax.ShapeDtypeStruct((NCORE, N, 6 * H), jnp.float32),
    )(x_real, x_imag, wsel)


_sc_mesh = plsc.VectorSubcoreMesh(core_axis_name="c", subcore_axis_name="s")

_sc_params = pltpu.CompilerParams()
if "needs_layout_passes" in pltpu.CompilerParams.__dataclass_fields__:
    _sc_params = dataclasses.replace(_sc_params, needs_layout_passes=False)


@functools.partial(
    pl.kernel,
    out_type=jax.ShapeDtypeStruct((NCORE * N, C), jnp.float32),
    mesh=_sc_mesh,
    scratch_types=[
        pltpu.VMEM((B, 8), jnp.float32),
        pltpu.VMEM((B, 8), jnp.float32),
        pltpu.VMEM((B, 8), jnp.float32),
        pltpu.VMEM((B, 8), jnp.float32),
        pltpu.VMEM((B,), jnp.int32),
        pltpu.VMEM((B,), jnp.int32),
        pltpu.VMEM((B,), jnp.int32),
        pltpu.VMEM((B,), jnp.int32),
        pltpu.VMEM((B,), jnp.int32),
        pltpu.VMEM((B,), jnp.int32),
        pltpu.VMEM((B, 6 * H), jnp.float32),
        pltpu.VMEM((B, 6 * H), jnp.float32),
        pltpu.VMEM((B, C), jnp.float32),
        pltpu.VMEM((B, C), jnp.float32),
        pltpu.VMEM((ZROW, C), jnp.float32),
        pltpu.VMEM_SHARED((N, C), jnp.float32),
        pltpu.SemaphoreType.DMA,
        pltpu.SemaphoreType.DMA,
        pltpu.SemaphoreType.DMA,
        pltpu.SemaphoreType.DMA,
        pltpu.SemaphoreType.DMA,
    ],
    compiler_params=_sc_params,
)
def _sc_spmm(ycat_hbm, rec_hbm, src_hbm, dst_hbm, out_hbm,
             recv0, recv1, recv2, recv3, srcv0, srcv1,
             dstv0, dstv1, dstv2, dstv3,
             rows0, rows1, outv0, outv1, zbuf, acc,
             gsem0, gsem1, ssem0, ssem1, msem):
    c = lax.axis_index("c")
    s = lax.axis_index("s")

    recvs = [recv0, recv1, recv2, recv3]
    srcvs = [srcv0, srcv1]
    dstvs = [dstv0, dstv1, dstv2, dstv3]
    rowss = [rows0, rows1]
    outvs = [outv0, outv1]
    gsems = [gsem0, gsem1]
    ssems = [ssem0, ssem1]

    zero16 = jnp.zeros((LANES,), jnp.float32)

    @pl.loop(0, ZROW)
    def _(r):
        for kk in range(C // LANES):
            zbuf[r, pl.ds(kk * LANES, LANES)] = zero16

    @pl.loop(s, NZCH, step=NSUB)
    def _(ch):
        pltpu.sync_copy(zbuf, acc.at[pl.ds(ch * ZROW, ZROW)])

    plsc.subcore_barrier()

    icol = [jnp.full((LANES,), i, dtype=jnp.int32) for i in range(6)]
    base0 = s * EPS

    def meta_issue(t, recv_b, srcv_b, dstv_b):
        base = base0 + t * B
        pltpu.async_copy(src_hbm.at[pl.ds(c * E + base, B)], srcv_b, msem)
        pltpu.async_copy(dst_hbm.at[pl.ds(base, B)], dstv_b, msem)
        pltpu.async_copy(rec_hbm.at[pl.ds(base, B)], recv_b, msem)

    def meta_wait(t, recv_b, srcv_b, dstv_b):
        base = base0 + t * B
        pltpu.make_async_copy(src_hbm.at[pl.ds(c * E + base, B)], srcv_b, msem).wait()
        pltpu.make_async_copy(dst_hbm.at[pl.ds(base, B)], dstv_b, msem).wait()
        pltpu.make_async_copy(rec_hbm.at[pl.ds(base, B)], recv_b, msem).wait()

    def gather_issue(srcv_b, rows_b, sem):
        pltpu.async_copy(ycat_hbm.at[srcv_b], rows_b, sem)

    def gather_wait(srcv_b, rows_b, sem):
        pltpu.make_async_copy(ycat_hbm.at[srcv_b], rows_b, sem).wait()

    def scatter_issue(outv_b, dstv_b, sem):
        pltpu.async_copy(outv_b, acc.at[dstv_b], sem, add=True)

    def scatter_wait(outv_b, dstv_b, sem):
        pltpu.make_async_copy(outv_b, acc.at[dstv_b], sem).wait()

    def compute(recv_b, rows_b, outv_b):
        @plsc.parallel_loop(0, B, 1, unroll=2)
        def _(j):
            jvec = jnp.full((LANES,), j, dtype=jnp.int32)
            w0 = plsc.load_gather(recv_b, [jvec, icol[0]])
            w1 = plsc.load_gather(recv_b, [jvec, icol[1]])
            w2 = plsc.load_gather(recv_b, [jvec, icol[2]])
            w3 = plsc.load_gather(recv_b, [jvec, icol[3]])
            w4 = plsc.load_gather(recv_b, [jvec, icol[4]])
            w5 = plsc.load_gather(recv_b, [jvec, icol[5]])
            for k in range(H // LANES):
                o = k * LANES
                yr0 = rows_b[j, pl.ds(o, LANES)]
                yr1 = rows_b[j, pl.ds(H + o, LANES)]
                yr2 = rows_b[j, pl.ds(2 * H + o, LANES)]
                yi0 = rows_b[j, pl.ds(3 * H + o, LANES)]
                yi1 = rows_b[j, pl.ds(4 * H + o, LANES)]
                yi2 = rows_b[j, pl.ds(5 * H + o, LANES)]
                re = (w0 * yr0 + w1 * yr1 + w2 * yr2
                      - w3 * yi0 - w4 * yi1 - w5 * yi2)
                im = (w3 * yr0 + w4 * yr1 + w5 * yr2
                      + w0 * yi0 + w1 * yi1 + w2 * yi2)
                outv_b[j, pl.ds(o, LANES)] = re
                outv_b[j, pl.ds(H + o, LANES)] = im

    for t in range(2):
        meta_issue(t, recvs[t], srcvs[t], dstvs[t])
        meta_wait(t, recvs[t], srcvs[t], dstvs[t])
        gather_issue(srcvs[t], rowss[t], gsems[t])

    @pl.loop(0, NCHUNK - 1, step=4)
    def _(t0):
        for b4 in range(4):
            t = t0 + b4
            b2 = b4 % 2
            gather_wait(srcvs[b2], rowss[b2], gsems[b2])

            @pl.when(t >= 2)
            def _():
                scatter_wait(outvs[b2], dstvs[(b4 + 2) % 4], ssems[b2])

            meta_issue(t + 2, recvs[(b4 + 2) % 4], srcvs[b2],
                       dstvs[(b4 + 2) % 4])
            compute(recvs[b4], rowss[b2], outvs[b2])
            scatter_issue(outvs[b2], dstvs[b4], ssems[b2])
            meta_wait(t + 2, recvs[(b4 + 2) % 4], srcvs[b2],
                      dstvs[(b4 + 2) % 4])
            gather_issue(srcvs[b2], rowss[b2], gsems[b2])

    gather_wait(srcvs[0], rowss[0], gsems[0])
    scatter_wait(outvs[0], dstvs[2], ssems[0])
    compute(recvs[0], rowss[0], outvs[0])
    scatter_issue(outvs[0], dstvs[0], ssems[0])
    gather_wait(srcvs[1], rowss[1], gsems[1])
    scatter_wait(outvs[1], dstvs[3], ssems[1])
    scatter_wait(outvs[0], dstvs[0], ssems[0])

    plsc.subcore_barrier()

    @pl.loop(s, NDCH, step=NSUB)
    def _(ch):
        r0 = ch * DROW
        pltpu.sync_copy(acc.at[pl.ds(r0, DROW)],
                        out_hbm.at[pl.ds(c * N + r0, DROW)])


def kernel(x_real, x_imag, edge_index, l_real_w, l_imag_w, weight, bias):
    dst = edge_index[0]
    src = edge_index[1]

    wsel = jnp.stack([
        jnp.concatenate([weight[0][:, :H], weight[1][:, :H], weight[2][:, :H]], axis=1),
        jnp.concatenate([weight[0][:, H:], weight[1][:, H:], weight[2][:, H:]], axis=1),
    ])

    ycat = _project(x_real, x_imag, wsel).reshape(NCORE * N, 6 * H)

    rec = jnp.concatenate(
        [l_real_w.T, l_imag_w.T, jnp.zeros((E, 2), jnp.float32)], axis=1)
    rec = jnp.concatenate([rec, jnp.zeros((B, 8), jnp.float32)], axis=0)
    src2 = jnp.concatenate([src, src + N, jnp.zeros((B,), jnp.int32)])
    dst2 = jnp.concatenate([dst, jnp.zeros((B,), jnp.int32)])

    out = _sc_spmm(ycat, rec, src2, dst2).reshape(NCORE, N, C)

    real = jnp.concatenate([out[0, :, :H], out[1, :, :H]], axis=1) + bias
    imag = jnp.concatenate([out[0, :, H:], out[1, :, H:]], axis=1) + bias
    return (real, imag)


# speedup vs baseline: 1.8558x; 1.2765x over previous
import dataclasses
import functools

import jax
import jax.numpy as jnp
from jax import lax
from jax.experimental import pallas as pl
from jax.experimental.pallas import tpu as pltpu
from jax.experimental.pallas import tpu_sc as plsc

N = 10000
E = 320000
C = 128
H = 64
NCORE = 2
NSUB = 16
LANES = 16

EPS = E // NSUB
B = 32
NCHUNK = EPS // B
ZROW = 8
NZCH = N // ZROW
DROW = 40
NDCH = N // DROW

MBLK = 400


def _ycat_body(xr_ref, xi_ref, w_ref, o_ref):
    w = w_ref[0]
    a = jnp.dot(xr_ref[...], w, preferred_element_type=jnp.float32)
    b = jnp.dot(xi_ref[...], w, preferred_element_type=jnp.float32)
    o_ref[0, :, 0:192] = a
    o_ref[0, :, 192:384] = b


def _project(x_real, x_imag, wsel):
    return pl.pallas_call(
        _ycat_body,
        grid=(NCORE, N // MBLK),
        in_specs=[
            pl.BlockSpec((MBLK, C), lambda c, n: (n, 0)),
            pl.BlockSpec((MBLK, C), lambda c, n: (n, 0)),
            pl.BlockSpec((1, C, 3 * H), lambda c, n: (c, 0, 0)),
        ],
        out_specs=pl.BlockSpec((1, MBLK, 6 * H), lambda c, n: (c, n, 0)),
        out_shape=jax.ShapeDtypeStruct((NCORE, N, 6 * H), jnp.float32),
    )(x_real, x_imag, wsel)


_sc_mesh = plsc.VectorSubcoreMesh(core_axis_name="c", subcore_axis_name="s")

_sc_params = pltpu.CompilerParams()
if "needs_layout_passes" in pltpu.CompilerParams.__dataclass_fields__:
    _sc_params = dataclasses.replace(_sc_params, needs_layout_passes=False)


@functools.partial(
    pl.kernel,
    out_type=jax.ShapeDtypeStruct((NCORE * N, C), jnp.float32),
    mesh=_sc_mesh,
    scratch_types=[
        pltpu.VMEM((6, B), jnp.float32),
        pltpu.VMEM((6, B), jnp.float32),
        pltpu.VMEM((6, B), jnp.float32),
        pltpu.VMEM((6, B), jnp.float32),
        pltpu.VMEM((B,), jnp.int32),
        pltpu.VMEM((B,), jnp.int32),
        pltpu.VMEM((B,), jnp.int32),
        pltpu.VMEM((B,), jnp.int32),
        pltpu.VMEM((B,), jnp.int32),
        pltpu.VMEM((B,), jnp.int32),
        pltpu.VMEM((B, 6 * H), jnp.float32),
        pltpu.VMEM((B, 6 * H), jnp.float32),
        pltpu.VMEM((B, C), jnp.float32),
        pltpu.VMEM((B, C), jnp.float32),
        pltpu.VMEM((ZROW, C), jnp.float32),
        pltpu.VMEM_SHARED((N, C), jnp.float32),
        pltpu.SemaphoreType.DMA,
        pltpu.SemaphoreType.DMA,
        pltpu.SemaphoreType.DMA,
        pltpu.SemaphoreType.DMA,
        pltpu.SemaphoreType.DMA,
    ],
    compiler_params=_sc_params,
)
def _sc_spmm(ycat_hbm, lr_hbm, li_hbm, src_hbm, dst_hbm, out_hbm,
             recv0, recv1, recv2, recv3, srcv0, srcv1,
             dstv0, dstv1, dstv2, dstv3,
             rows0, rows1, outv0, outv1, zbuf, acc,
             gsem0, gsem1, ssem0, ssem1, msem):
    c = lax.axis_index("c")
    s = lax.axis_index("s")

    recvs = [recv0, recv1, recv2, recv3]
    srcvs = [srcv0, srcv1]
    dstvs = [dstv0, dstv1, dstv2, dstv3]
    rowss = [rows0, rows1]
    outvs = [outv0, outv1]
    gsems = [gsem0, gsem1]
    ssems = [ssem0, ssem1]

    zero16 = jnp.zeros((LANES,), jnp.float32)

    @pl.loop(0, ZROW)
    def _(r):
        for kk in range(C // LANES):
            zbuf[r, pl.ds(kk * LANES, LANES)] = zero16

    @pl.loop(s, NZCH, step=NSUB)
    def _(ch):
        pltpu.sync_copy(zbuf, acc.at[pl.ds(ch * ZROW, ZROW)])

    plsc.subcore_barrier()

    icol = [jnp.full((LANES,), i, dtype=jnp.int32) for i in range(6)]
    coffv = jnp.full((LANES,), c * N, dtype=jnp.int32)
    base0 = s * EPS

    def _base(t):
        return jnp.minimum(base0 + t * B, E - B)

    def meta_issue(t, recv_b, srcv_b, dstv_b):
        base = _base(t)
        pltpu.async_copy(src_hbm.at[pl.ds(base, B)], srcv_b, msem)
        pltpu.async_copy(dst_hbm.at[pl.ds(base, B)], dstv_b, msem)
        for i in range(3):
            pltpu.async_copy(lr_hbm.at[i, pl.ds(base, B)], recv_b.at[i], msem)
            pltpu.async_copy(li_hbm.at[i, pl.ds(base, B)], recv_b.at[i + 3], msem)

    def meta_wait(t, recv_b, srcv_b, dstv_b):
        base = _base(t)
        pltpu.make_async_copy(src_hbm.at[pl.ds(base, B)], srcv_b, msem).wait()
        pltpu.make_async_copy(dst_hbm.at[pl.ds(base, B)], dstv_b, msem).wait()
        for i in range(3):
            pltpu.make_async_copy(lr_hbm.at[i, pl.ds(base, B)], recv_b.at[i], msem).wait()
            pltpu.make_async_copy(li_hbm.at[i, pl.ds(base, B)], recv_b.at[i + 3], msem).wait()
        for hh in range(B // LANES):
            srcv_b[pl.ds(hh * LANES, LANES)] = (
                srcv_b[pl.ds(hh * LANES, LANES)] + coffv)

    def gather_issue(srcv_b, rows_b, sem):
        pltpu.async_copy(ycat_hbm.at[srcv_b], rows_b, sem)

    def gather_wait(srcv_b, rows_b, sem):
        pltpu.make_async_copy(ycat_hbm.at[srcv_b], rows_b, sem).wait()

    def scatter_issue(outv_b, dstv_b, sem):
        pltpu.async_copy(outv_b, acc.at[dstv_b], sem, add=True)

    def scatter_wait(outv_b, dstv_b, sem):
        pltpu.make_async_copy(outv_b, acc.at[dstv_b], sem).wait()

    def compute(recv_b, rows_b, outv_b):
        @plsc.parallel_loop(0, B, 1, unroll=2)
        def _(j):
            jvec = jnp.full((LANES,), j, dtype=jnp.int32)
            w0 = plsc.load_gather(recv_b, [icol[0], jvec])
            w1 = plsc.load_gather(recv_b, [icol[1], jvec])
            w2 = plsc.load_gather(recv_b, [icol[2], jvec])
            w3 = plsc.load_gather(recv_b, [icol[3], jvec])
            w4 = plsc.load_gather(recv_b, [icol[4], jvec])
            w5 = plsc.load_gather(recv_b, [icol[5], jvec])
            for k in range(H // LANES):
                o = k * LANES
                yr0 = rows_b[j, pl.ds(o, LANES)]
                yr1 = rows_b[j, pl.ds(H + o, LANES)]
                yr2 = rows_b[j, pl.ds(2 * H + o, LANES)]
                yi0 = rows_b[j, pl.ds(3 * H + o, LANES)]
                yi1 = rows_b[j, pl.ds(4 * H + o, LANES)]
                yi2 = rows_b[j, pl.ds(5 * H + o, LANES)]
                re = (w0 * yr0 + w1 * yr1 + w2 * yr2
                      - w3 * yi0 - w4 * yi1 - w5 * yi2)
                im = (w3 * yr0 + w4 * yr1 + w5 * yr2
                      + w0 * yi0 + w1 * yi1 + w2 * yi2)
                outv_b[j, pl.ds(o, LANES)] = re
                outv_b[j, pl.ds(H + o, LANES)] = im

    for t in range(2):
        meta_issue(t, recvs[t], srcvs[t], dstvs[t])
        meta_wait(t, recvs[t], srcvs[t], dstvs[t])
        gather_issue(srcvs[t], rowss[t], gsems[t])

    @pl.loop(0, NCHUNK - 1, step=4)
    def _(t0):
        for b4 in range(4):
            t = t0 + b4
            b2 = b4 % 2
            gather_wait(srcvs[b2], rowss[b2], gsems[b2])

            @pl.when(t >= 2)
            def _():
                scatter_wait(outvs[b2], dstvs[(b4 + 2) % 4], ssems[b2])

            meta_issue(t + 2, recvs[(b4 + 2) % 4], srcvs[b2],
                       dstvs[(b4 + 2) % 4])
            compute(recvs[b4], rowss[b2], outvs[b2])
            scatter_issue(outvs[b2], dstvs[b4], ssems[b2])
            meta_wait(t + 2, recvs[(b4 + 2) % 4], srcvs[b2],
                      dstvs[(b4 + 2) % 4])
            gather_issue(srcvs[b2], rowss[b2], gsems[b2])

    gather_wait(srcvs[0], rowss[0], gsems[0])
    scatter_wait(outvs[0], dstvs[2], ssems[0])
    compute(recvs[0], rowss[0], outvs[0])
    scatter_issue(outvs[0], dstvs[0], ssems[0])
    gather_wait(srcvs[1], rowss[1], gsems[1])
    scatter_wait(outvs[1], dstvs[3], ssems[1])
    scatter_wait(outvs[0], dstvs[0], ssems[0])

    plsc.subcore_barrier()

    @pl.loop(s, NDCH, step=NSUB)
    def _(ch):
        r0 = ch * DROW
        pltpu.sync_copy(acc.at[pl.ds(r0, DROW)],
                        out_hbm.at[pl.ds(c * N + r0, DROW)])


def kernel(x_real, x_imag, edge_index, l_real_w, l_imag_w, weight, bias):
    dst = edge_index[0]
    src = edge_index[1]

    wsel = jnp.stack([
        jnp.concatenate([weight[0][:, :H], weight[1][:, :H], weight[2][:, :H]], axis=1),
        jnp.concatenate([weight[0][:, H:], weight[1][:, H:], weight[2][:, H:]], axis=1),
    ])

    ycat = _project(x_real, x_imag, wsel).reshape(NCORE * N, 6 * H)

    out = _sc_spmm(ycat, l_real_w, l_imag_w, src, dst).reshape(NCORE, N, C)

    real = jnp.concatenate([out[0, :, :H], out[1, :, :H]], axis=1) + bias
    imag = jnp.concatenate([out[0, :, H:], out[1, :, H:]], axis=1) + bias
    return (real, imag)
